# baseline (speedup 1.0000x reference)
import functools

import jax
import jax.numpy as jnp
from jax import lax
from jax.experimental import pallas as pl
from jax.experimental.pallas import tpu as pltpu
from jax.experimental.pallas import tpu_sc as plsc

N = 10000
E = 320000
D = 128
NS = 16

CHUNK = 64
NSPLIT = 4
HC = CHUNK // NSPLIT
NCHUNKS = 320
PER_TILE = CHUNK * NCHUNKS
EPAD = NS * PER_TILE
NPAD = 10112
ZROWS = NPAD // NS


def _sc_body(table, src3, dst3, urep, zg, gh_out,
             acc, src00, src01, src02, src03, src10, src11, src12, src13,
             dst0, dst1, uv0, uv1,
             rows0, rows1, isem0, isem1, gsem0, gsem1):
    c = lax.axis_index("c")
    s = lax.axis_index("s")

    pltpu.sync_copy(zg, acc.at[pl.ds(s * ZROWS, ZROWS)])

    plsc.subcore_barrier()

    srcb = ((src00, src01, src02, src03), (src10, src11, src12, src13))
    dstb = (dst0, dst1)
    uvb = (uv0, uv1)
    rowsb = (rows0, rows1)
    isem = (isem0, isem1)
    gsem = (gsem0, gsem1)

    def start_idx(k, b):
        for p in range(NSPLIT):
            pltpu.async_copy(src3.at[s, k, pl.ds(p * HC, HC)], srcb[b][p], isem[b])
        pltpu.async_copy(dst3.at[s, k], dstb[b], isem[b])

        @pl.when(c == 1)
        def _():
            pltpu.async_copy(urep.at[s, pl.ds(k * CHUNK, CHUNK)], uvb[b], isem[b])

    def wait_idx(b):
        for p in range(NSPLIT):
            pltpu.make_async_copy(src3.at[s, 0, pl.ds(0, HC)], srcb[b][p], isem[b]).wait()
        pltpu.make_async_copy(dst3.at[s, 0], dstb[b], isem[b]).wait()

        @pl.when(c == 1)
        def _():
            pltpu.make_async_copy(urep.at[s, pl.ds(0, CHUNK)], uvb[b], isem[b]).wait()

    def start_gather(b):
        for p in range(NSPLIT):
            pltpu.async_copy(table.at[srcb[b][p]], rowsb[b].at[pl.ds(p * HC, HC)], gsem[b])

    def wait_gather(b):
        for p in range(NSPLIT):
            pltpu.make_async_copy(table.at[srcb[b][p]], rowsb[b].at[pl.ds(p * HC, HC)], gsem[b]).wait()

    def process(b):
        @pl.when(c == 1)
        def _():
            @plsc.parallel_loop(0, CHUNK, 1, unroll=4)
            def _(j):
                uj = uvb[b][j, :]
                for q in range(8):
                    sl = pl.ds(q * 16, 16)
                    rowsb[b][j, sl] = rowsb[b][j, sl] * uj

        pltpu.sync_copy(rowsb[b], acc.at[dstb[b]], add=True)

    start_idx(0, 0)
    start_idx(1, 1)
    wait_idx(0)
    start_gather(0)

    def pair_body(g, _):
        k0 = 2 * g
        wait_idx(1)
        start_gather(1)
        wait_gather(0)
        process(0)

        @pl.when(k0 + 2 < NCHUNKS)
        def _():
            start_idx(k0 + 2, 0)
            wait_idx(0)
            start_gather(0)

        wait_gather(1)
        process(1)

        @pl.when(k0 + 3 < NCHUNKS)
        def _():
            start_idx(k0 + 3, 1)
        return 0

    lax.fori_loop(0, NCHUNKS // 2, pair_body, 0)

    plsc.subcore_barrier()

    r0 = s * ZROWS
    pltpu.sync_copy(acc.at[pl.ds(r0, ZROWS)], gh_out.at[c, pl.ds(r0, ZROWS)])


def _make_sc():
    outs = jax.ShapeDtypeStruct((2, NPAD, D), jnp.float32)
    scratch = [
        pltpu.VMEM_SHARED((NPAD, D), jnp.float32),
        pltpu.VMEM((HC,), jnp.int32),
        pltpu.VMEM((HC,), jnp.int32),
        pltpu.VMEM((HC,), jnp.int32),
        pltpu.VMEM((HC,), jnp.int32),
        pltpu.VMEM((HC,), jnp.int32),
        pltpu.VMEM((HC,), jnp.int32),
        pltpu.VMEM((HC,), jnp.int32),
        pltpu.VMEM((HC,), jnp.int32),
        pltpu.VMEM((CHUNK,), jnp.int32),
        pltpu.VMEM((CHUNK,), jnp.int32),
        pltpu.VMEM((CHUNK, 16), jnp.float32),
        pltpu.VMEM((CHUNK, 16), jnp.float32),
        pltpu.VMEM((CHUNK, D), jnp.float32),
        pltpu.VMEM((CHUNK, D), jnp.float32),
        pltpu.SemaphoreType.DMA,
        pltpu.SemaphoreType.DMA,
        pltpu.SemaphoreType.DMA,
        pltpu.SemaphoreType.DMA,
    ]
    mesh = plsc.VectorSubcoreMesh(core_axis_name="c", subcore_axis_name="s")
    return pl.kernel(
        _sc_body,
        out_type=outs,
        mesh=mesh,
        scratch_types=scratch,
    )


def _cnt_body(dst3, zg, cnt_out, acccnt, dstc, onesv):
    c = lax.axis_index("c")
    s = lax.axis_index("s")

    pltpu.sync_copy(zg, acccnt.at[pl.ds(s * ZROWS, ZROWS)])

    def ones_body(j, _):
        def col_body(q, _):
            onesv[j, pl.ds(q * 16, 16)] = jnp.ones((16,), jnp.float32)
            return 0
        lax.fori_loop(0, D // 16, col_body, 0)
        return 0
    lax.fori_loop(0, CHUNK, ones_body, 0)

    plsc.subcore_barrier()

    half = NCHUNKS // 2

    def chunk_body(k, _):
        pltpu.sync_copy(dst3.at[s, c * half + k], dstc)
        pltpu.sync_copy(onesv, acccnt.at[dstc], add=True)
        return 0

    lax.fori_loop(0, half, chunk_body, 0)

    plsc.subcore_barrier()

    r0 = s * ZROWS
    pltpu.sync_copy(acccnt.at[pl.ds(r0, ZROWS)], cnt_out.at[c, pl.ds(r0, ZROWS)])


def _make_cnt():
    scratch = [
        pltpu.VMEM_SHARED((NPAD, D), jnp.float32),
        pltpu.VMEM((CHUNK,), jnp.int32),
        pltpu.VMEM((CHUNK, D), jnp.float32),
    ]
    mesh = plsc.VectorSubcoreMesh(core_axis_name="c", subcore_axis_name="s")
    return pl.kernel(
        _cnt_body,
        out_type=jax.ShapeDtypeStruct((2, NPAD, D), jnp.float32),
        mesh=mesh,
        scratch_types=scratch,
    )


def _dense_body(mode, h, g, ha, cnt, W, root, bias, o):
    W0 = W[0]
    W1 = W[1]
    Wd = W1 - W0
    f32 = jnp.float32
    pre = (
        jnp.dot(g[0], W0, preferred_element_type=f32)
        + jnp.dot(ha[0], Wd, preferred_element_type=f32)
    )
    c = jnp.maximum(cnt[0, :, 0:1] + cnt[1, :, 0:1], 1.0)
    r = pre / c + jnp.dot(h[...], root[...], preferred_element_type=f32) + bias[...]
    if mode < 2:
        r = r * jax.nn.sigmoid(r)
        o[...] = r
    else:
        m = jnp.max(r, axis=1, keepdims=True)
        e = jnp.exp(r - m)
        sm = jnp.sum(e, axis=1, keepdims=True)
        o[...] = r - m - jnp.log(sm)


def _dense(h, gh, cnt, W, root, bias, mode):
    Do = W.shape[2]
    R = 1000
    nb = N // R
    return pl.pallas_call(
        functools.partial(_dense_body, mode),
        grid=(nb,),
        in_specs=[
            pl.BlockSpec((R, D), lambda i: (i, 0)),
            pl.BlockSpec((1, R, D), lambda i: (0, i, 0)),
            pl.BlockSpec((1, R, D), lambda i: (1, i, 0)),
            pl.BlockSpec((2, R, D), lambda i: (0, i, 0)),
            pl.BlockSpec((2, D, Do), lambda i: (0, 0, 0)),
            pl.BlockSpec((D, Do), lambda i: (0, 0)),
            pl.BlockSpec((1, Do), lambda i: (0, 0)),
        ],
        out_specs=pl.BlockSpec((R, Do), lambda i: (i, 0)),
        out_shape=jax.ShapeDtypeStruct((N, Do), jnp.float32),
    )(h, gh, gh, cnt, W, root, bias)


def kernel(x, edge_index, pseudo, W0, root0, bias0, W1, root1, bias1, W2, root2, bias2):
    src = edge_index[0]
    dst = edge_index[1]
    u = pseudo[:, 0]

    pad = EPAD - E
    srcp = jnp.concatenate([src, jnp.zeros((pad,), jnp.int32)])
    dstp = jnp.concatenate([dst, jnp.full((pad,), N, jnp.int32)])
    up = jnp.concatenate([u, jnp.zeros((pad,), jnp.float32)])
    src3 = srcp.reshape(NS, NCHUNKS, CHUNK)
    dst3 = dstp.reshape(NS, NCHUNKS, CHUNK)
    urep = jnp.broadcast_to(up[:, None], (EPAD, 16)).reshape(NS, NCHUNKS * CHUNK, 16)
    zg = jnp.zeros((ZROWS, D), jnp.float32)

    sc = _make_sc()
    cnt = _make_cnt()(dst3, zg)

    gh = sc(x, src3, dst3, urep, zg)
    h = _dense(x, gh, cnt, W0, root0, bias0.reshape(1, -1), mode=0)
    gh = sc(h, src3, dst3, urep, zg)
    h = _dense(h, gh, cnt, W1, root1, bias1.reshape(1, -1), mode=1)
    gh = sc(h, src3, dst3, urep, zg)
    return _dense(h, gh, cnt, W2, root2, bias2.reshape(1, -1), mode=2)


# speedup vs baseline: 1.2138x; 1.2138x over previous
import functools

import jax
import jax.numpy as jnp
from jax import lax
from jax.experimental import pallas as pl
from jax.experimental.pallas import tpu as pltpu
from jax.experimental.pallas import tpu_sc as plsc

N = 10000
E = 320000
D = 128
NS = 16

CHUNK = 48
NSPLIT = 2
HC = CHUNK // NSPLIT
NCHUNKS = 422
PER_TILE = CHUNK * NCHUNKS
EPAD = NS * PER_TILE
NPAD = 10112
ZROWS = NPAD // NS


def _sc_body(table, src3, dst3, urep, zg, gh_out,
             acc, src00, src01, src10, src11, src20, src21,
             dst0, dst1, dst2, uv0, uv1, uv2,
             rows0, rows1, rows2,
             isem0, isem1, isem2, gsem0, gsem1, gsem2):
    c = lax.axis_index("c")
    s = lax.axis_index("s")

    pltpu.sync_copy(zg, acc.at[pl.ds(s * ZROWS, ZROWS)])

    plsc.subcore_barrier()

    srcb = ((src00, src01), (src10, src11), (src20, src21))
    dstb = (dst0, dst1, dst2)
    uvb = (uv0, uv1, uv2)
    rowsb = (rows0, rows1, rows2)
    isem = (isem0, isem1, isem2)
    gsem = (gsem0, gsem1, gsem2)

    def start_idx(k, b):
        for p in range(NSPLIT):
            pltpu.async_copy(src3.at[s, k, pl.ds(p * HC, HC)], srcb[b][p], isem[b])
        pltpu.async_copy(dst3.at[s, k], dstb[b], isem[b])

        @pl.when(c == 1)
        def _():
            pltpu.async_copy(urep.at[s, pl.ds(k * CHUNK, CHUNK)], uvb[b], isem[b])

    def wait_idx(b):
        for p in range(NSPLIT):
            pltpu.make_async_copy(src3.at[s, 0, pl.ds(0, HC)], srcb[b][p], isem[b]).wait()
        pltpu.make_async_copy(dst3.at[s, 0], dstb[b], isem[b]).wait()

        @pl.when(c == 1)
        def _():
            pltpu.make_async_copy(urep.at[s, pl.ds(0, CHUNK)], uvb[b], isem[b]).wait()

    def start_gather(b):
        for p in range(NSPLIT):
            pltpu.async_copy(table.at[srcb[b][p]], rowsb[b].at[pl.ds(p * HC, HC)], gsem[b])

    def wait_gather(b):
        for p in range(NSPLIT):
            pltpu.make_async_copy(table.at[srcb[b][p]], rowsb[b].at[pl.ds(p * HC, HC)], gsem[b]).wait()

    def process(b):
        @pl.when(c == 1)
        def _():
            @plsc.parallel_loop(0, CHUNK, 1, unroll=4)
            def _(j):
                uj = uvb[b][j, :]
                for q in range(8):
                    sl = pl.ds(q * 16, 16)
                    rowsb[b][j, sl] = rowsb[b][j, sl] * uj

        pltpu.sync_copy(rowsb[b], acc.at[dstb[b]], add=True)

    start_idx(0, 0)
    start_idx(1, 1)
    start_idx(2, 2)
    wait_idx(0)
    start_gather(0)
    wait_idx(1)
    start_gather(1)

    def tri_body(g, _):
        k0 = 3 * g
        for r in range(3):
            b = r
            bn = (r + 2) % 3
            k = k0 + r
            wait_gather(b)
            wait_idx(bn)
            start_gather(bn)
            process(b)

            @pl.when(k + 3 < NCHUNKS)
            def _():
                start_idx(k + 3, b)
        return 0

    lax.fori_loop(0, (NCHUNKS - 2) // 3, tri_body, 0)

    wait_gather(0)
    process(0)
    wait_gather(1)
    process(1)

    plsc.subcore_barrier()

    r0 = s * ZROWS
    pltpu.sync_copy(acc.at[pl.ds(r0, ZROWS)], gh_out.at[c, pl.ds(r0, ZROWS)])


def _make_sc():
    outs = jax.ShapeDtypeStruct((2, NPAD, D), jnp.float32)
    scratch = [
        pltpu.VMEM_SHARED((NPAD, D), jnp.float32),
        pltpu.VMEM((HC,), jnp.int32),
        pltpu.VMEM((HC,), jnp.int32),
        pltpu.VMEM((HC,), jnp.int32),
        pltpu.VMEM((HC,), jnp.int32),
        pltpu.VMEM((HC,), jnp.int32),
        pltpu.VMEM((HC,), jnp.int32),
        pltpu.VMEM((CHUNK,), jnp.int32),
        pltpu.VMEM((CHUNK,), jnp.int32),
        pltpu.VMEM((CHUNK,), jnp.int32),
        pltpu.VMEM((CHUNK, 16), jnp.float32),
        pltpu.VMEM((CHUNK, 16), jnp.float32),
        pltpu.VMEM((CHUNK, 16), jnp.float32),
        pltpu.VMEM((CHUNK, D), jnp.float32),
        pltpu.VMEM((CHUNK, D), jnp.float32),
        pltpu.VMEM((CHUNK, D), jnp.float32),
        pltpu.SemaphoreType.DMA,
        pltpu.SemaphoreType.DMA,
        pltpu.SemaphoreType.DMA,
        pltpu.SemaphoreType.DMA,
        pltpu.SemaphoreType.DMA,
        pltpu.SemaphoreType.DMA,
    ]
    mesh = plsc.VectorSubcoreMesh(core_axis_name="c", subcore_axis_name="s")
    return pl.kernel(
        _sc_body,
        out_type=outs,
        mesh=mesh,
        scratch_types=scratch,
    )


def _cnt_body(dst3, zg, cnt_out, acccnt, dstc, onesv):
    c = lax.axis_index("c")
    s = lax.axis_index("s")

    pltpu.sync_copy(zg, acccnt.at[pl.ds(s * ZROWS, ZROWS)])

    def ones_body(j, _):
        def col_body(q, _):
            onesv[j, pl.ds(q * 16, 16)] = jnp.ones((16,), jnp.float32)
            return 0
        lax.fori_loop(0, D // 16, col_body, 0)
        return 0
    lax.fori_loop(0, CHUNK, ones_body, 0)

    plsc.subcore_barrier()

    half = NCHUNKS // 2

    def chunk_body(k, _):
        pltpu.sync_copy(dst3.at[s, c * half + k], dstc)
        pltpu.sync_copy(onesv, acccnt.at[dstc], add=True)
        return 0

    lax.fori_loop(0, half, chunk_body, 0)

    plsc.subcore_barrier()

    r0 = s * ZROWS
    pltpu.sync_copy(acccnt.at[pl.ds(r0, ZROWS)], cnt_out.at[c, pl.ds(r0, ZROWS)])


def _make_cnt():
    scratch = [
        pltpu.VMEM_SHARED((NPAD, D), jnp.float32),
        pltpu.VMEM((CHUNK,), jnp.int32),
        pltpu.VMEM((CHUNK, D), jnp.float32),
    ]
    mesh = plsc.VectorSubcoreMesh(core_axis_name="c", subcore_axis_name="s")
    return pl.kernel(
        _cnt_body,
        out_type=jax.ShapeDtypeStruct((2, NPAD, D), jnp.float32),
        mesh=mesh,
        scratch_types=scratch,
    )


def _dense_body(mode, h, g, ha, cnt, W, root, bias, o):
    W0 = W[0]
    W1 = W[1]
    Wd = W1 - W0
    f32 = jnp.float32
    pre = (
        jnp.dot(g[0], W0, preferred_element_type=f32)
        + jnp.dot(ha[0], Wd, preferred_element_type=f32)
    )
    c = jnp.maximum(cnt[0, :, 0:1] + cnt[1, :, 0:1], 1.0)
    r = pre / c + jnp.dot(h[...], root[...], preferred_element_type=f32) + bias[...]
    if mode < 2:
        r = r * jax.nn.sigmoid(r)
        o[...] = r
    else:
        m = jnp.max(r, axis=1, keepdims=True)
        e = jnp.exp(r - m)
        sm = jnp.sum(e, axis=1, keepdims=True)
        o[...] = r - m - jnp.log(sm)


def _dense(h, gh, cnt, W, root, bias, mode):
    Do = W.shape[2]
    R = 1000
    nb = N // R
    return pl.pallas_call(
        functools.partial(_dense_body, mode),
        grid=(nb,),
        in_specs=[
            pl.BlockSpec((R, D), lambda i: (i, 0)),
            pl.BlockSpec((1, R, D), lambda i: (0, i, 0)),
            pl.BlockSpec((1, R, D), lambda i: (1, i, 0)),
            pl.BlockSpec((2, R, D), lambda i: (0, i, 0)),
            pl.BlockSpec((2, D, Do), lambda i: (0, 0, 0)),
            pl.BlockSpec((D, Do), lambda i: (0, 0)),
            pl.BlockSpec((1, Do), lambda i: (0, 0)),
        ],
        out_specs=pl.BlockSpec((R, Do), lambda i: (i, 0)),
        out_shape=jax.ShapeDtypeStruct((N, Do), jnp.float32),
    )(h, gh, gh, cnt, W, root, bias)


def kernel(x, edge_index, pseudo, W0, root0, bias0, W1, root1, bias1, W2, root2, bias2):
    src = edge_index[0]
    dst = edge_index[1]
    u = pseudo[:, 0]

    pad = EPAD - E
    srcp = jnp.concatenate([src, jnp.zeros((pad,), jnp.int32)])
    dstp = jnp.concatenate([dst, jnp.full((pad,), N, jnp.int32)])
    up = jnp.concatenate([u, jnp.zeros((pad,), jnp.float32)])
    src3 = srcp.reshape(NS, NCHUNKS, CHUNK)
    dst3 = dstp.reshape(NS, NCHUNKS, CHUNK)
    urep = jnp.broadcast_to(up[:, None], (EPAD, 16)).reshape(NS, NCHUNKS * CHUNK, 16)
    zg = jnp.zeros((ZROWS, D), jnp.float32)

    sc = _make_sc()
    cnt = _make_cnt()(dst3, zg)

    gh = sc(x, src3, dst3, urep, zg)
    h = _dense(x, gh, cnt, W0, root0, bias0.reshape(1, -1), mode=0)
    gh = sc(h, src3, dst3, urep, zg)
    h = _dense(h, gh, cnt, W1, root1, bias1.reshape(1, -1), mode=1)
    gh = sc(h, src3, dst3, urep, zg)
    return _dense(h, gh, cnt, W2, root2, bias2.reshape(1, -1), mode=2)


# speedup vs baseline: 1.3371x; 1.1016x over previous
import functools

import jax
import jax.numpy as jnp
from jax import lax
from jax.experimental import pallas as pl
from jax.experimental.pallas import tpu as pltpu
from jax.experimental.pallas import tpu_sc as plsc

N = 10000
E = 320000
D = 128
NS = 16

CHUNK = 48
NSPLIT = 2
HC = CHUNK // NSPLIT
NCHUNKS = 422
PER_TILE = CHUNK * NCHUNKS
EPAD = NS * PER_TILE
NPAD = 10112
ZROWS = NPAD // NS


def _sc_body(table, src3, dst3, urep, zg, gh_out,
             acc, src00, src01, src10, src11, src20, src21,
             dst0, dst1, dst2, uv0, uv1, uv2,
             rows0, rows1, rows2,
             isem0, isem1, isem2, gsem0, gsem1, gsem2,
             ssem0, ssem1, ssem2, dsem0, dsem1, dsem2):
    c = lax.axis_index("c")
    s = lax.axis_index("s")

    pltpu.sync_copy(zg, acc.at[pl.ds(s * ZROWS, ZROWS)])

    plsc.subcore_barrier()

    srcb = ((src00, src01), (src10, src11), (src20, src21))
    dstb = (dst0, dst1, dst2)
    uvb = (uv0, uv1, uv2)
    rowsb = (rows0, rows1, rows2)
    isem = (isem0, isem1, isem2)
    gsem = (gsem0, gsem1, gsem2)
    dsem = (dsem0, dsem1, dsem2)

    def start_idx(k, b):
        for p in range(NSPLIT):
            pltpu.async_copy(src3.at[s, k, pl.ds(p * HC, HC)], srcb[b][p], isem[b])

        @pl.when(c == 1)
        def _():
            pltpu.async_copy(urep.at[s, pl.ds(k * CHUNK, CHUNK)], uvb[b], isem[b])

    def wait_idx(b):
        for p in range(NSPLIT):
            pltpu.make_async_copy(src3.at[s, 0, pl.ds(0, HC)], srcb[b][p], isem[b]).wait()

        @pl.when(c == 1)
        def _():
            pltpu.make_async_copy(urep.at[s, pl.ds(0, CHUNK)], uvb[b], isem[b]).wait()

    def start_dst(k, b):
        pltpu.async_copy(dst3.at[s, k], dstb[b], dsem[b])

    def wait_dst(b):
        pltpu.make_async_copy(dst3.at[s, 0], dstb[b], dsem[b]).wait()

    def start_gather(b):
        for p in range(NSPLIT):
            pltpu.async_copy(table.at[srcb[b][p]], rowsb[b].at[pl.ds(p * HC, HC)], gsem[b])

    def wait_gather(b):
        for p in range(NSPLIT):
            pltpu.make_async_copy(table.at[srcb[b][p]], rowsb[b].at[pl.ds(p * HC, HC)], gsem[b]).wait()

    ssem = (ssem0, ssem1, ssem2)

    def scale(b):
        @pl.when(c == 1)
        def _():
            @plsc.parallel_loop(0, CHUNK, 1, unroll=4)
            def _(j):
                uj = uvb[b][j, :]
                for q in range(8):
                    sl = pl.ds(q * 16, 16)
                    rowsb[b][j, sl] = rowsb[b][j, sl] * uj

    def start_scatter(b):
        pltpu.async_copy(rowsb[b], acc.at[dstb[b]], ssem[b], add=True)

    def wait_scatter(b):
        pltpu.make_async_copy(rowsb[b], acc.at[dstb[b]], ssem[b]).wait()

    start_idx(0, 0)
    start_idx(1, 1)
    start_idx(2, 2)
    start_dst(0, 0)
    start_dst(1, 1)
    wait_idx(0)
    start_gather(0)
    wait_idx(1)
    start_gather(1)

    def tri_body(g, _):
        k0 = 3 * g
        for r in range(3):
            b = r
            bn = (r + 2) % 3
            k = k0 + r
            wait_gather(b)
            wait_idx(bn)

            @pl.when(k > 0)
            def _():
                wait_scatter(bn)

            @pl.when(k + 2 < NCHUNKS)
            def _():
                start_dst(k + 2, bn)

            start_gather(bn)
            scale(b)
            wait_dst(b)
            start_scatter(b)

            @pl.when(k + 3 < NCHUNKS)
            def _():
                start_idx(k + 3, b)
        return 0

    lax.fori_loop(0, (NCHUNKS - 2) // 3, tri_body, 0)

    wait_gather(0)
    wait_scatter(2)
    scale(0)
    wait_dst(0)
    start_scatter(0)
    wait_gather(1)
    wait_scatter(0)
    scale(1)
    wait_dst(1)
    start_scatter(1)
    wait_scatter(1)

    plsc.subcore_barrier()

    r0 = s * ZROWS
    pltpu.sync_copy(acc.at[pl.ds(r0, ZROWS)], gh_out.at[c, pl.ds(r0, ZROWS)])


def _make_sc():
    outs = jax.ShapeDtypeStruct((2, NPAD, D), jnp.float32)
    scratch = [
        pltpu.VMEM_SHARED((NPAD, D), jnp.float32),
        pltpu.VMEM((HC,), jnp.int32),
        pltpu.VMEM((HC,), jnp.int32),
        pltpu.VMEM((HC,), jnp.int32),
        pltpu.VMEM((HC,), jnp.int32),
        pltpu.VMEM((HC,), jnp.int32),
        pltpu.VMEM((HC,), jnp.int32),
        pltpu.VMEM((CHUNK,), jnp.int32),
        pltpu.VMEM((CHUNK,), jnp.int32),
        pltpu.VMEM((CHUNK,), jnp.int32),
        pltpu.VMEM((CHUNK, 16), jnp.float32),
        pltpu.VMEM((CHUNK, 16), jnp.float32),
        pltpu.VMEM((CHUNK, 16), jnp.float32),
        pltpu.VMEM((CHUNK, D), jnp.float32),
        pltpu.VMEM((CHUNK, D), jnp.float32),
        pltpu.VMEM((CHUNK, D), jnp.float32),
        pltpu.SemaphoreType.DMA,
        pltpu.SemaphoreType.DMA,
        pltpu.SemaphoreType.DMA,
        pltpu.SemaphoreType.DMA,
        pltpu.SemaphoreType.DMA,
        pltpu.SemaphoreType.DMA,
        pltpu.SemaphoreType.DMA,
        pltpu.SemaphoreType.DMA,
        pltpu.SemaphoreType.DMA,
        pltpu.SemaphoreType.DMA,
        pltpu.SemaphoreType.DMA,
        pltpu.SemaphoreType.DMA,
    ]
    mesh = plsc.VectorSubcoreMesh(core_axis_name="c", subcore_axis_name="s")
    return pl.kernel(
        _sc_body,
        out_type=outs,
        mesh=mesh,
        scratch_types=scratch,
    )


def _cnt_body(dst3, zg, cnt_out, acccnt, dstc, onesv):
    c = lax.axis_index("c")
    s = lax.axis_index("s")

    pltpu.sync_copy(zg, acccnt.at[pl.ds(s * ZROWS, ZROWS)])

    def ones_body(j, _):
        def col_body(q, _):
            onesv[j, pl.ds(q * 16, 16)] = jnp.ones((16,), jnp.float32)
            return 0
        lax.fori_loop(0, D // 16, col_body, 0)
        return 0
    lax.fori_loop(0, CHUNK, ones_body, 0)

    plsc.subcore_barrier()

    half = NCHUNKS // 2

    def chunk_body(k, _):
        pltpu.sync_copy(dst3.at[s, c * half + k], dstc)
        pltpu.sync_copy(onesv, acccnt.at[dstc], add=True)
        return 0

    lax.fori_loop(0, half, chunk_body, 0)

    plsc.subcore_barrier()

    r0 = s * ZROWS
    pltpu.sync_copy(acccnt.at[pl.ds(r0, ZROWS)], cnt_out.at[c, pl.ds(r0, ZROWS)])


def _make_cnt():
    scratch = [
        pltpu.VMEM_SHARED((NPAD, D), jnp.float32),
        pltpu.VMEM((CHUNK,), jnp.int32),
        pltpu.VMEM((CHUNK, D), jnp.float32),
    ]
    mesh = plsc.VectorSubcoreMesh(core_axis_name="c", subcore_axis_name="s")
    return pl.kernel(
        _cnt_body,
        out_type=jax.ShapeDtypeStruct((2, NPAD, D), jnp.float32),
        mesh=mesh,
        scratch_types=scratch,
    )


def _dense_body(mode, h, g, ha, cnt, W, root, bias, o):
    W0 = W[0]
    W1 = W[1]
    Wd = W1 - W0
    f32 = jnp.float32
    pre = (
        jnp.dot(g[0], W0, preferred_element_type=f32)
        + jnp.dot(ha[0], Wd, preferred_element_type=f32)
    )
    c = jnp.maximum(cnt[0, :, 0:1] + cnt[1, :, 0:1], 1.0)
    r = pre / c + jnp.dot(h[...], root[...], preferred_element_type=f32) + bias[...]
    if mode < 2:
        r = r * jax.nn.sigmoid(r)
        o[...] = r
    else:
        m = jnp.max(r, axis=1, keepdims=True)
        e = jnp.exp(r - m)
        sm = jnp.sum(e, axis=1, keepdims=True)
        o[...] = r - m - jnp.log(sm)


def _dense(h, gh, cnt, W, root, bias, mode):
    Do = W.shape[2]
    R = 1000
    nb = N // R
    return pl.pallas_call(
        functools.partial(_dense_body, mode),
        grid=(nb,),
        in_specs=[
            pl.BlockSpec((R, D), lambda i: (i, 0)),
            pl.BlockSpec((1, R, D), lambda i: (0, i, 0)),
            pl.BlockSpec((1, R, D), lambda i: (1, i, 0)),
            pl.BlockSpec((2, R, D), lambda i: (0, i, 0)),
            pl.BlockSpec((2, D, Do), lambda i: (0, 0, 0)),
            pl.BlockSpec((D, Do), lambda i: (0, 0)),
            pl.BlockSpec((1, Do), lambda i: (0, 0)),
        ],
        out_specs=pl.BlockSpec((R, Do), lambda i: (i, 0)),
        out_shape=jax.ShapeDtypeStruct((N, Do), jnp.float32),
    )(h, gh, gh, cnt, W, root, bias)


def kernel(x, edge_index, pseudo, W0, root0, bias0, W1, root1, bias1, W2, root2, bias2):
    src = edge_index[0]
    dst = edge_index[1]
    u = pseudo[:, 0]

    pad = EPAD - E
    srcp = jnp.concatenate([src, jnp.zeros((pad,), jnp.int32)])
    dstp = jnp.concatenate([dst, jnp.full((pad,), N, jnp.int32)])
    up = jnp.concatenate([u, jnp.zeros((pad,), jnp.float32)])
    src3 = srcp.reshape(NS, NCHUNKS, CHUNK)
    dst3 = dstp.reshape(NS, NCHUNKS, CHUNK)
    urep = jnp.broadcast_to(up[:, None], (EPAD, 16)).reshape(NS, NCHUNKS * CHUNK, 16)
    zg = jnp.zeros((ZROWS, D), jnp.float32)

    sc = _make_sc()
    cnt = _make_cnt()(dst3, zg)

    gh = sc(x, src3, dst3, urep, zg)
    h = _dense(x, gh, cnt, W0, root0, bias0.reshape(1, -1), mode=0)
    gh = sc(h, src3, dst3, urep, zg)
    h = _dense(h, gh, cnt, W1, root1, bias1.reshape(1, -1), mode=1)
    gh = sc(h, src3, dst3, urep, zg)
    return _dense(h, gh, cnt, W2, root2, bias2.reshape(1, -1), mode=2)
